# baseline (speedup 1.0000x reference)
import functools
import math

import jax
import jax.numpy as jnp
from jax import lax
from jax.experimental import pallas as pl
from jax.experimental.pallas import tpu as pltpu
from jax.experimental.pallas import tpu_sc as plsc

NC = 2
NS = 16
NW = NC * NS
LANES = 16
CHUNK = 128
NBUF = 3
SUB = 8


TVB = 336


@functools.lru_cache(maxsize=None)
def _build_transpose(V, D):
    vpt = (V // NW) // 8 * 8
    vtail = V - vpt * NW
    nck = vpt // TVB
    assert nck * TVB == vpt and vtail % 8 == 0
    mesh = plsc.VectorSubcoreMesh(
        core_axis_name="c", subcore_axis_name="s",
        num_cores=NC, num_subcores=NS)

    @functools.partial(
        pl.kernel,
        out_type=jax.ShapeDtypeStruct((V, D), jnp.float32),
        mesh=mesh,
        scratch_types=[
            pltpu.VMEM((2, D, TVB), jnp.float32),
            pltpu.VMEM((2, TVB, D + 1), jnp.float32),
            pltpu.SemaphoreType.DMA((2,)),
            pltpu.SemaphoreType.DMA((2,)),
        ],
        compiler_params=pltpu.CompilerParams(
            use_tc_tiling_on_sc=False, needs_layout_passes=False),
    )
    def tr_kernel(tt_hbm, out_hbm, ibuf, obuf, gsem, ssem):
        wid = lax.axis_index("s") * NC + lax.axis_index("c")
        v0 = wid * vpt
        base_iota = lax.iota(jnp.int32, LANES)
        rv = [base_iota + vg * LANES for vg in range(TVB // LANES)]

        def start_read(c, b):
            pltpu.make_async_copy(
                tt_hbm.at[:, pl.ds(v0 + c * TVB, TVB)], ibuf.at[b],
                gsem.at[b]).start()

        def wait_read(b):
            pltpu.make_async_copy(
                tt_hbm.at[:, pl.ds(0, TVB)], ibuf.at[b], gsem.at[b]).wait()

        def start_write(c, b):
            pltpu.make_async_copy(
                obuf.at[b, :, pl.ds(0, D)],
                out_hbm.at[pl.ds(v0 + c * TVB, TVB)], ssem.at[b]).start()

        def wait_write(b):
            pltpu.make_async_copy(
                obuf.at[b, :, pl.ds(0, D)],
                out_hbm.at[pl.ds(0, TVB)], ssem.at[b]).wait()

        def transpose_chunk(b):
            @pl.loop(0, D)
            def _f(f):
                cv = jnp.zeros((LANES,), jnp.int32) + f
                for vg in range(TVB // LANES):
                    val = ibuf[b, f, pl.ds(vg * LANES, LANES)]
                    plsc.store_scatter(obuf.at[b], [rv[vg], cv], val)

        for b in range(2):
            start_read(b, b)
        for c in range(2):
            b = c % 2
            wait_read(b)
            transpose_chunk(b)
            start_write(c, b)
            start_read(c + 2, b)

        @pl.loop(2, nck - (nck % 2), step=2)
        def _grp(g):
            for b in range(2):
                c = g + b
                wait_read(b)
                wait_write(b)
                transpose_chunk(b)
                start_write(c, b)

                @pl.when(c + 2 < nck)
                def _():
                    start_read(c + 2, b)

        for c in range(nck - (nck % 2), nck):
            b = c % 2
            wait_read(b)
            wait_write(b)
            transpose_chunk(b)
            start_write(c, b)
            if c + 2 < nck:
                start_read(c + 2, b)
        for b in range(2):
            wait_write(b)

        if vtail:
            @pl.when(wid == NW - 1)
            def _tail():
                pltpu.sync_copy(
                    tt_hbm.at[:, pl.ds(V - vtail, vtail)],
                    ibuf.at[0, :, pl.ds(0, vtail)])

                @pl.loop(0, D)
                def _f(f):
                    cv = jnp.zeros((LANES,), jnp.int32) + f
                    for vg in range(vtail // LANES):
                        val = ibuf[0, f, pl.ds(vg * LANES, LANES)]
                        plsc.store_scatter(obuf.at[0], [rv[vg], cv], val)
                pltpu.sync_copy(
                    obuf.at[0, pl.ds(0, vtail), pl.ds(0, D)],
                    out_hbm.at[pl.ds(V - vtail, vtail)])

    return tr_kernel


@functools.lru_cache(maxsize=None)
def _build(BL, L, V, D, scale):
    B = BL // L
    n_chunks = BL // CHUNK
    cpt = n_chunks // NW
    cb_per_l = B // CHUNK
    cb_shift = cb_per_l.bit_length() - 1
    assert cb_per_l == 1 << cb_shift and cpt > 2 * NBUF
    mesh = plsc.VectorSubcoreMesh(
        core_axis_name="c", subcore_axis_name="s",
        num_cores=NC, num_subcores=NS)

    @functools.partial(
        pl.kernel,
        out_type=jax.ShapeDtypeStruct(
            (L, D // SUB, B // CHUNK, SUB, CHUNK), jnp.float32),
        mesh=mesh,
        scratch_types=[
            pltpu.VMEM((cpt, CHUNK), jnp.int32),
            pltpu.VMEM((NBUF, CHUNK, D), jnp.float32),
            pltpu.VMEM((NBUF, SUB, SUB, CHUNK + 1), jnp.float32),
            pltpu.SemaphoreType.DMA((NBUF,)),
            pltpu.SemaphoreType.DMA((NBUF,)),
        ],
        compiler_params=pltpu.CompilerParams(
            use_tc_tiling_on_sc=False, needs_layout_passes=False),
    )
    def emb_kernel(idx_hbm, tab_hbm, out_hbm, idx_v, gbuf, sbuf, gsem, ssem):
        wid = lax.axis_index("s") * NC + lax.axis_index("c")
        c0 = wid * cpt
        pltpu.sync_copy(idx_hbm.at[pl.ds(c0, cpt)], idx_v)
        base_iota = lax.iota(jnp.int32, LANES)
        ones = jnp.ones((LANES,), jnp.int32)
        fh = [jax.lax.shift_right_logical(base_iota + g * LANES, 3)
              for g in range(D // LANES)]
        fl = [jax.lax.bitwise_and(base_iota + g * LANES, 7)
              for g in range(D // LANES)]

        def prep_gather(j, b):
            pltpu.make_async_copy(
                tab_hbm.at[idx_v.at[j]], gbuf.at[b], gsem.at[b]).start()

        def wait_gather(b):
            pltpu.make_async_copy(
                tab_hbm.at[idx_v.at[0]], gbuf.at[b], gsem.at[b]).wait()

        def start_scatter(j, b):
            c = c0 + j
            l = jax.lax.shift_right_logical(c, cb_shift)
            cb = jax.lax.bitwise_and(c, cb_per_l - 1)
            pltpu.make_async_copy(
                sbuf.at[b, :, :, pl.ds(0, CHUNK)],
                out_hbm.at[l, :, cb], ssem.at[b]).start()

        def wait_scatter(b):
            pltpu.make_async_copy(
                sbuf.at[b, :, :, pl.ds(0, CHUNK)],
                out_hbm.at[0, :, 0], ssem.at[b]).wait()

        def compute_chunk(j, b):
            ng = D // LANES

            @pl.loop(0, CHUNK, step=2, unroll=4, init_carry=base_iota * 0)
            def _row(jj, col):
                colb = col + ones
                va = [gbuf[b, jj, pl.ds(g * LANES, LANES)]
                      for g in range(ng)]
                vb = [gbuf[b, jj + 1, pl.ds(g * LANES, LANES)]
                      for g in range(ng)]
                sa = [v * scale for v in va]
                sb = [v * scale for v in vb]
                for g in range(ng):
                    plsc.store_scatter(
                        sbuf.at[b], [fh[g], fl[g], col], sa[g])
                for g in range(ng):
                    plsc.store_scatter(
                        sbuf.at[b], [fh[g], fl[g], colb], sb[g])
                return col + 2

        n_main = NBUF + (cpt - 2 * NBUF) // NBUF * NBUF
        for b in range(NBUF):
            prep_gather(b, b)
        for j in range(NBUF):
            b = j % NBUF
            wait_gather(b)
            compute_chunk(j, b)
            start_scatter(j, b)
            prep_gather(j + NBUF, b)

        @pl.loop(NBUF, n_main, step=NBUF)
        def _group(g):
            for b in range(NBUF):
                j = g + b
                wait_gather(b)
                wait_scatter(b)
                compute_chunk(j, b)
                start_scatter(j, b)

                @pl.when(j + NBUF < cpt)
                def _():
                    prep_gather(j + NBUF, b)

        for j in range(n_main, cpt):
            b = j % NBUF
            wait_gather(b)
            wait_scatter(b)
            compute_chunk(j, b)
            start_scatter(j, b)
            if j + NBUF < cpt:
                prep_gather(j + NBUF, b)

        for b in range(NBUF):
            wait_scatter(b)

    return emb_kernel


def kernel(x, table):
    V, D = table.shape
    Bb, L = x.shape
    scale = math.sqrt(D)
    tab2 = _build_transpose(V, D)(table.T)
    idx = x.T.astype(jnp.int32).reshape((Bb * L) // CHUNK, CHUNK)
    out5 = _build(Bb * L, L, V, D, scale)(idx, tab2)
    return out5.transpose(2, 4, 0, 1, 3).reshape(Bb, L, D)


# speedup vs baseline: 7.2758x; 7.2758x over previous
import functools
import math

import jax
import jax.numpy as jnp
from jax import lax
from jax.experimental import pallas as pl
from jax.experimental.pallas import tpu as pltpu
from jax.experimental.pallas import tpu_sc as plsc

NC = 2
NS = 16
NW = NC * NS
LANES = 16
CHUNK = 128
NBUF = 3
SUB = 8


@functools.lru_cache(maxsize=None)
def _build(BL, L, V, D, scale):
    B = BL // L
    n_chunks = BL // CHUNK
    cpt = n_chunks // NW
    cb_per_l = B // CHUNK
    cb_shift = cb_per_l.bit_length() - 1
    assert cb_per_l == 1 << cb_shift and cpt > 2 * NBUF
    mesh = plsc.VectorSubcoreMesh(
        core_axis_name="c", subcore_axis_name="s",
        num_cores=NC, num_subcores=NS)

    @functools.partial(
        pl.kernel,
        out_type=jax.ShapeDtypeStruct(
            (L, D // SUB, B // CHUNK, SUB, CHUNK), jnp.float32),
        mesh=mesh,
        scratch_types=[
            pltpu.VMEM((cpt, CHUNK), jnp.int32),
            pltpu.VMEM((NBUF, CHUNK, D), jnp.float32),
            pltpu.VMEM((NBUF, SUB, SUB, CHUNK + 1), jnp.float32),
            pltpu.SemaphoreType.DMA((NBUF,)),
            pltpu.SemaphoreType.DMA((NBUF,)),
        ],
        compiler_params=pltpu.CompilerParams(
            use_tc_tiling_on_sc=False, needs_layout_passes=False),
    )
    def emb_kernel(idx_hbm, tab_hbm, out_hbm, idx_v, gbuf, sbuf, gsem, ssem):
        wid = lax.axis_index("s") * NC + lax.axis_index("c")
        c0 = wid * cpt
        pltpu.sync_copy(idx_hbm.at[pl.ds(c0, cpt)], idx_v)
        base_iota = lax.iota(jnp.int32, LANES)
        ones = jnp.ones((LANES,), jnp.int32)
        fh = [jax.lax.shift_right_logical(base_iota + g * LANES, 3)
              for g in range(D // LANES)]
        fl = [jax.lax.bitwise_and(base_iota + g * LANES, 7)
              for g in range(D // LANES)]

        def prep_gather(j, b):
            pltpu.make_async_copy(
                tab_hbm.at[idx_v.at[j]], gbuf.at[b], gsem.at[b]).start()

        def wait_gather(b):
            pltpu.make_async_copy(
                tab_hbm.at[idx_v.at[0]], gbuf.at[b], gsem.at[b]).wait()

        def start_scatter(j, b):
            c = c0 + j
            l = jax.lax.shift_right_logical(c, cb_shift)
            cb = jax.lax.bitwise_and(c, cb_per_l - 1)
            pltpu.make_async_copy(
                sbuf.at[b, :, :, pl.ds(0, CHUNK)],
                out_hbm.at[l, :, cb], ssem.at[b]).start()

        def wait_scatter(b):
            pltpu.make_async_copy(
                sbuf.at[b, :, :, pl.ds(0, CHUNK)],
                out_hbm.at[0, :, 0], ssem.at[b]).wait()

        def compute_chunk(j, b):
            ng = D // LANES

            @pl.loop(0, CHUNK, step=2, unroll=4, init_carry=base_iota * 0)
            def _row(jj, col):
                colb = col + ones
                va = [gbuf[b, jj, pl.ds(g * LANES, LANES)]
                      for g in range(ng)]
                vb = [gbuf[b, jj + 1, pl.ds(g * LANES, LANES)]
                      for g in range(ng)]
                sa = [v * scale for v in va]
                sb = [v * scale for v in vb]
                for g in range(ng):
                    plsc.store_scatter(
                        sbuf.at[b], [fh[g], fl[g], col], sa[g])
                for g in range(ng):
                    plsc.store_scatter(
                        sbuf.at[b], [fh[g], fl[g], colb], sb[g])
                return col + 2

        n_main = NBUF + (cpt - 2 * NBUF) // NBUF * NBUF
        for b in range(NBUF):
            prep_gather(b, b)
        for j in range(NBUF):
            b = j % NBUF
            wait_gather(b)
            compute_chunk(j, b)
            start_scatter(j, b)
            prep_gather(j + NBUF, b)

        @pl.loop(NBUF, n_main, step=NBUF)
        def _group(g):
            for b in range(NBUF):
                j = g + b
                wait_gather(b)
                wait_scatter(b)
                compute_chunk(j, b)
                start_scatter(j, b)

                @pl.when(j + NBUF < cpt)
                def _():
                    prep_gather(j + NBUF, b)

        for j in range(n_main, cpt):
            b = j % NBUF
            wait_gather(b)
            wait_scatter(b)
            compute_chunk(j, b)
            start_scatter(j, b)
            if j + NBUF < cpt:
                prep_gather(j + NBUF, b)

        for b in range(NBUF):
            wait_scatter(b)

    return emb_kernel


def kernel(x, table):
    V, D = table.shape
    Bb, L = x.shape
    scale = math.sqrt(D)
    tab2 = jnp.pad(table, ((0, 0), (0, D))).reshape(2 * V, D)
    idx = (x.T.astype(jnp.int32) * 2).reshape((Bb * L) // CHUNK, CHUNK)
    out5 = _build(Bb * L, L, V, D, scale)(idx, tab2)
    return out5.transpose(2, 4, 0, 1, 3).reshape(Bb, L, D)
